# baseline (speedup 1.0000x reference)
import functools

import jax
import jax.numpy as jnp
import numpy as np
from jax import lax
from jax.experimental import pallas as pl
from jax.experimental.pallas import tpu as pltpu
from jax.experimental.pallas import tpu_sc as plsc

EMBED_DIM = 64
CHUNK = 128
NBUF = 4
PAD_H = 24

VOCAB = 1000000
NBLK = (VOCAB + 127) // 128
TAIL_COL = 128 * (VOCAB // 128)
MAIN_BLKS = 244


def _transpose_block(slab, outbuf, row_ids, par_ids, n_d=64):
    @plsc.parallel_loop(0, n_d, unroll=2)
    def body(d):
        for grp in range(len(row_ids)):
            vec = slab[d, pl.ds(16 * grp, 16)]
            plsc.store_scatter(outbuf, [row_ids[grp], par_ids + d], vec)


@functools.lru_cache(maxsize=None)
def _build_transpose():
    mesh = plsc.VectorSubcoreMesh(core_axis_name="c", subcore_axis_name="s")
    info = plsc.get_sparse_core_info()
    nc = info.num_cores

    @functools.partial(
        pl.kernel,
        out_type=jax.ShapeDtypeStruct((VOCAB // 2, 2 * EMBED_DIM), jnp.float32),
        mesh=mesh,
        compiler_params=pltpu.CompilerParams(
            use_tc_tiling_on_sc=True, needs_layout_passes=False),
        scratch_types=[
            *[pltpu.VMEM((EMBED_DIM, 128), jnp.float32) for _ in range(2)],
            *[pltpu.VMEM((EMBED_DIM, 2 * EMBED_DIM), jnp.float32) for _ in range(2)],
            pltpu.VMEM((EMBED_DIM, EMBED_DIM), jnp.float32),
            *[pltpu.SemaphoreType.DMA for _ in range(4)],
        ],
    )
    def k(tt_hbm, out_hbm, slab0, slab1, ob0, ob1, tail_slab, gi0, gi1, go0, go1):
        slabs = (slab0, slab1)
        obufs = (ob0, ob1)
        gin = (gi0, gi1)
        gout = (go0, go1)
        wid = lax.axis_index("s") * nc + lax.axis_index("c")
        lane = lax.iota(jnp.int32, 16)
        row_ids = [(16 * grp + lane) // 2 for grp in range(8)]
        par_ids = 64 * (lane % 2)

        def blk_of(k_):
            return wid + 32 * k_

        def start_in(k_, slot):
            pltpu.make_async_copy(
                tt_hbm.at[:, pl.ds(pl.multiple_of(blk_of(k_) * 128, 128), 128)],
                slabs[slot], gin[slot]).start()

        def wait_in(k_, slot):
            pltpu.make_async_copy(
                tt_hbm.at[:, pl.ds(pl.multiple_of(blk_of(k_) * 128, 128), 128)],
                slabs[slot], gin[slot]).wait()

        def start_out(k_, slot):
            pltpu.make_async_copy(
                obufs[slot], out_hbm.at[pl.ds(pl.multiple_of(blk_of(k_) * 64, 64), 64)],
                gout[slot]).start()

        def wait_out(k_, slot):
            pltpu.make_async_copy(
                obufs[slot], out_hbm.at[pl.ds(pl.multiple_of(blk_of(k_) * 64, 64), 64)],
                gout[slot]).wait()

        start_in(0, 0)
        start_in(1, 1)
        for kk in (0, 1):
            wait_in(kk, kk)
            _transpose_block(slabs[kk], obufs[kk], row_ids, par_ids)
            start_out(kk, kk)
            start_in(kk + 2, kk)

        def body(ii, carry):
            for slot in (0, 1):
                k_ = 2 * ii + slot
                wait_in(k_, slot)
                wait_out(k_ - 2, slot)
                _transpose_block(slabs[slot], obufs[slot], row_ids, par_ids)
                start_out(k_, slot)
                start_in(k_ + 2, slot)
            return carry

        lax.fori_loop(1, 121, body, 0)

        for kk in (242, 243):
            slot = kk % 2
            wait_in(kk, slot)
            wait_out(kk - 2, slot)
            _transpose_block(slabs[slot], obufs[slot], row_ids, par_ids)
            start_out(kk, slot)
        for kk in (242, 243):
            wait_out(kk, kk % 2)

        @pl.when(wid < 4)
        def _():
            blk = 32 * MAIN_BLKS + wid
            col0 = pl.multiple_of(blk * 128, 128)
            pltpu.sync_copy(tt_hbm.at[:, pl.ds(col0, 128)], slabs[0])
            _transpose_block(slabs[0], obufs[0], row_ids, par_ids)
            pltpu.sync_copy(
                obufs[0], out_hbm.at[pl.ds(pl.multiple_of(blk * 64, 64), 64)])

        @pl.when(wid == 4)
        def _():
            pltpu.sync_copy(tt_hbm.at[:, pl.ds(TAIL_COL, 64)], tail_slab)
            _transpose_block(tail_slab, obufs[0], row_ids[:4], par_ids)
            pltpu.sync_copy(
                obufs[0].at[pl.ds(0, 32)],
                out_hbm.at[pl.ds(TAIL_COL // 2, 32)])

    return k


@functools.lru_cache(maxsize=None)
def _build_gather(n_rows, hist):
    info = plsc.get_sparse_core_info()
    nc, ns = info.num_cores, info.num_subcores
    nw = nc * ns
    per_w = n_rows // nw
    assert per_w * nw == n_rows and per_w % CHUNK == 0
    n_chunks = per_w // CHUNK
    assert n_chunks % NBUF == 0
    rounds = n_chunks // NBUF
    n_batches = n_rows // hist
    out_rows = n_batches * PAD_H * 2
    mesh = plsc.VectorSubcoreMesh(core_axis_name="c", subcore_axis_name="s")

    @functools.partial(
        pl.kernel,
        out_type=jax.ShapeDtypeStruct((out_rows, EMBED_DIM), jnp.float32),
        mesh=mesh,
        compiler_params=pltpu.CompilerParams(use_tc_tiling_on_sc=False),
        scratch_types=[
            pltpu.VMEM((n_chunks, CHUNK), jnp.int32),
            pltpu.VMEM((n_chunks, CHUNK), jnp.int32),
            *[pltpu.VMEM((CHUNK, EMBED_DIM), jnp.float32) for _ in range(NBUF)],
            *[pltpu.SemaphoreType.DMA for _ in range(2 * NBUF)],
        ],
    )
    def k(idx_hbm, didx_hbm, table_hbm, out_hbm, idx_v, didx_v, *rest):
        bufs = rest[:NBUF]
        gsems = rest[NBUF:2 * NBUF]
        ssems = rest[2 * NBUF:]
        wid = lax.axis_index("s") * nc + lax.axis_index("c")
        crow0 = wid * n_chunks

        pltpu.sync_copy(idx_hbm.at[pl.ds(crow0, n_chunks)], idx_v)
        pltpu.sync_copy(didx_hbm.at[pl.ds(crow0, n_chunks)], didx_v)

        def gather(g, b):
            return pltpu.make_async_copy(
                table_hbm.at[idx_v.at[g]], bufs[b], gsems[b])

        def scatter(g, b):
            return pltpu.make_async_copy(
                bufs[b], out_hbm.at[didx_v.at[g]], ssems[b])

        for b in range(NBUF):
            gather(b, b).start()

        def round_body(r, carry):
            for b in range(NBUF):
                g = r * NBUF + b
                gather(g, b).wait()
                scatter(g, b).start()
            for b in range(NBUF):
                g = r * NBUF + b
                scatter(g, b).wait()
                gather(g + NBUF, b).start()
            return carry

        lax.fori_loop(0, rounds - 1, round_body, 0)

        last = (rounds - 1) * NBUF
        for b in range(NBUF):
            gather(last + b, b).wait()
            scatter(last + b, b).start()
        for b in range(NBUF):
            scatter(last + b, b).wait()

    return k


def kernel(x, table):
    batch, hist = x.shape
    n_rows = batch * hist
    dense = _build_transpose()(table.T)
    t2 = dense.reshape(VOCAB, EMBED_DIM)
    idx = x.reshape(n_rows // CHUNK, CHUNK)
    n = np.arange(n_rows, dtype=np.int64)
    didx_np = 2 * (PAD_H * (n // hist) + n % hist)
    didx = jnp.asarray(
        didx_np.reshape(n_rows // CHUNK, CHUNK).astype(np.int32))
    out = _build_gather(n_rows, hist)(idx, didx, t2)
    out3 = out.reshape(batch, PAD_H, 2 * EMBED_DIM)
    return out3[:, :hist, :EMBED_DIM]


# speedup vs baseline: 1.3485x; 1.3485x over previous
import functools

import jax
import jax.numpy as jnp
import numpy as np
from jax import lax
from jax.experimental import pallas as pl
from jax.experimental.pallas import tpu as pltpu
from jax.experimental.pallas import tpu_sc as plsc

EMBED_DIM = 64
CHUNK = 128
NBUF = 4
PAD_H = 24


@functools.lru_cache(maxsize=None)
def _build(n_rows, hist):
    info = plsc.get_sparse_core_info()
    nc, ns = info.num_cores, info.num_subcores
    nw = nc * ns
    per_w = n_rows // nw
    assert per_w * nw == n_rows and per_w % CHUNK == 0
    n_chunks = per_w // CHUNK
    assert n_chunks % NBUF == 0
    rounds = n_chunks // NBUF
    n_batches = n_rows // hist
    out_rows = n_batches * PAD_H * 2
    mesh = plsc.VectorSubcoreMesh(core_axis_name="c", subcore_axis_name="s")

    @functools.partial(
        pl.kernel,
        out_type=jax.ShapeDtypeStruct((out_rows, EMBED_DIM), jnp.float32),
        mesh=mesh,
        compiler_params=pltpu.CompilerParams(use_tc_tiling_on_sc=False),
        scratch_types=[
            pltpu.VMEM((n_chunks, CHUNK), jnp.int32),
            pltpu.VMEM((n_chunks, CHUNK), jnp.int32),
            *[pltpu.VMEM((CHUNK, EMBED_DIM), jnp.float32) for _ in range(NBUF)],
            *[pltpu.SemaphoreType.DMA for _ in range(2 * NBUF)],
        ],
    )
    def k(idx_hbm, didx_hbm, table_hbm, out_hbm, idx_v, didx_v, *rest):
        bufs = rest[:NBUF]
        gsems = rest[NBUF:2 * NBUF]
        ssems = rest[2 * NBUF:]
        wid = lax.axis_index("s") * nc + lax.axis_index("c")
        crow0 = wid * n_chunks

        pltpu.sync_copy(idx_hbm.at[pl.ds(crow0, n_chunks)], idx_v)
        pltpu.sync_copy(didx_hbm.at[pl.ds(crow0, n_chunks)], didx_v)

        def gather(g, b):
            return pltpu.make_async_copy(
                table_hbm.at[idx_v.at[g]], bufs[b], gsems[b])

        def scatter(g, b):
            return pltpu.make_async_copy(
                bufs[b], out_hbm.at[didx_v.at[g]], ssems[b])

        for b in range(NBUF):
            gather(b, b).start()

        def round_body(r, carry):
            for b in range(NBUF):
                g = r * NBUF + b
                gather(g, b).wait()
                scatter(g, b).start()
            for b in range(NBUF):
                g = r * NBUF + b
                scatter(g, b).wait()
                gather(g + NBUF, b).start()
            return carry

        lax.fori_loop(0, rounds - 1, round_body, 0)

        last = (rounds - 1) * NBUF
        for b in range(NBUF):
            gather(last + b, b).wait()
            scatter(last + b, b).start()
        for b in range(NBUF):
            scatter(last + b, b).wait()

    return k


def kernel(x, table):
    batch, hist = x.shape
    n_rows = batch * hist
    idx = x.reshape(n_rows // CHUNK, CHUNK)
    n = np.arange(n_rows, dtype=np.int64)
    didx_np = 2 * (PAD_H * (n // hist) + n % hist)
    didx = jnp.asarray(
        didx_np.reshape(n_rows // CHUNK, CHUNK).astype(np.int32))
    out = _build(n_rows, hist)(idx, didx, table)
    out3 = out.reshape(batch, PAD_H, 2 * EMBED_DIM)
    return out3[:, :hist, :EMBED_DIM]


# speedup vs baseline: 1.3513x; 1.0021x over previous
import functools

import jax
import jax.numpy as jnp
import numpy as np
from jax import lax
from jax.experimental import pallas as pl
from jax.experimental.pallas import tpu as pltpu
from jax.experimental.pallas import tpu_sc as plsc

EMBED_DIM = 64
CHUNK = 128
NBUF = 8
PAD_H = 24


@functools.lru_cache(maxsize=None)
def _build(n_rows, hist):
    info = plsc.get_sparse_core_info()
    nc, ns = info.num_cores, info.num_subcores
    nw = nc * ns
    per_w = n_rows // nw
    assert per_w * nw == n_rows and per_w % CHUNK == 0
    n_chunks = per_w // CHUNK
    assert n_chunks % NBUF == 0
    rounds = n_chunks // NBUF
    n_batches = n_rows // hist
    out_rows = n_batches * PAD_H * 2
    mesh = plsc.VectorSubcoreMesh(core_axis_name="c", subcore_axis_name="s")

    @functools.partial(
        pl.kernel,
        out_type=jax.ShapeDtypeStruct((out_rows, EMBED_DIM), jnp.float32),
        mesh=mesh,
        compiler_params=pltpu.CompilerParams(use_tc_tiling_on_sc=False),
        scratch_types=[
            pltpu.VMEM((n_chunks, CHUNK), jnp.int32),
            pltpu.VMEM((n_chunks, CHUNK), jnp.int32),
            *[pltpu.VMEM((CHUNK, EMBED_DIM), jnp.float32) for _ in range(NBUF)],
            *[pltpu.SemaphoreType.DMA for _ in range(2 * NBUF)],
        ],
    )
    def k(idx_hbm, didx_hbm, table_hbm, out_hbm, idx_v, didx_v, *rest):
        bufs = rest[:NBUF]
        gsems = rest[NBUF:2 * NBUF]
        ssems = rest[2 * NBUF:]
        wid = lax.axis_index("s") * nc + lax.axis_index("c")
        crow0 = wid * n_chunks

        pltpu.sync_copy(idx_hbm.at[pl.ds(crow0, n_chunks)], idx_v)
        pltpu.sync_copy(didx_hbm.at[pl.ds(crow0, n_chunks)], didx_v)

        def gather(g, b):
            return pltpu.make_async_copy(
                table_hbm.at[idx_v.at[g]], bufs[b], gsems[b])

        def scatter(g, b):
            return pltpu.make_async_copy(
                bufs[b], out_hbm.at[didx_v.at[g]], ssems[b])

        for b in range(NBUF):
            gather(b, b).start()

        def round_body(r, carry):
            for b in range(NBUF):
                g = r * NBUF + b
                gather(g, b).wait()
                scatter(g, b).start()
            for b in range(NBUF):
                g = r * NBUF + b
                scatter(g, b).wait()
                gather(g + NBUF, b).start()
            return carry

        lax.fori_loop(0, rounds - 1, round_body, 0)

        last = (rounds - 1) * NBUF
        for b in range(NBUF):
            gather(last + b, b).wait()
            scatter(last + b, b).start()
        for b in range(NBUF):
            scatter(last + b, b).wait()

    return k


def kernel(x, table):
    batch, hist = x.shape
    n_rows = batch * hist
    idx = x.reshape(n_rows // CHUNK, CHUNK)
    n = np.arange(n_rows, dtype=np.int64)
    didx_np = 2 * (PAD_H * (n // hist) + n % hist)
    didx = jnp.asarray(
        didx_np.reshape(n_rows // CHUNK, CHUNK).astype(np.int32))
    out = _build(n_rows, hist)(idx, didx, table)
    out3 = out.reshape(batch, PAD_H, 2 * EMBED_DIM)
    return out3[:, :hist, :EMBED_DIM]
